# baseline (speedup 1.0000x reference)
import jax
import jax.numpy as jnp
from jax.experimental import pallas as pl
from jax.experimental.pallas import tpu as pltpu

_KB = 256
_K3 = 512


def _body(state_ref, w1_ref, b1_ref, w2_ref, b2_ref, w3_ref, b3_ref,
          v1_ref, vb1_ref, v2_ref, vb2_ref, v3_ref, vb3_ref, g_ref,
          probs_ref, value_ref, act_ref, alp_ref,
          sb, h1a, h2a, h1v, h2v, acca, accv, lg):
    B, S = state_ref.shape
    A = b3_ref.shape[1]
    nk = S // _KB
    n3 = S // _K3
    o2 = nk
    o3 = 2 * nk
    o4 = o3 + n3

    i = pl.program_id(0)

    @pl.when(i == 0)
    def _cast_state():
        sb[...] = state_ref[...].astype(jnp.bfloat16)

    def _pair(k, lhs_a, lhs_v, wa_ref, ba_ref, wv_ref, bv_ref, outa, outv):
        pa = jnp.dot(lhs_a[:, pl.ds(k * _KB, _KB)], wa_ref[...],
                     preferred_element_type=jnp.float32)
        pv = jnp.dot(lhs_v[:, pl.ds(k * _KB, _KB)], wv_ref[...],
                     preferred_element_type=jnp.float32)

        @pl.when(k == 0)
        def _():
            acca[...] = pa
            accv[...] = pv

        @pl.when(k > 0)
        def _():
            acca[...] = acca[...] + pa
            accv[...] = accv[...] + pv

        @pl.when(k == nk - 1)
        def _():
            outa[...] = jnp.tanh(acca[...] + ba_ref[...]).astype(jnp.bfloat16)
            outv[...] = jnp.tanh(accv[...] + bv_ref[...]).astype(jnp.bfloat16)

    @pl.when(i < o2)
    def _p0():
        _pair(i, sb, sb, w1_ref, b1_ref, v1_ref, vb1_ref, h1a, h1v)

    @pl.when((i >= o2) & (i < o3))
    def _p1():
        _pair(i - o2, h1a, h1v, w2_ref, b2_ref, v2_ref, vb2_ref, h2a, h2v)

    @pl.when((i >= o3) & (i < o4))
    def _p2():
        k = i - o3
        part = jnp.dot(h2a[:, pl.ds(k * _K3, _K3)], w3_ref[...],
                       preferred_element_type=jnp.float32)

        @pl.when(k == 0)
        def _():
            lg[...] = part

        @pl.when(k > 0)
        def _():
            lg[...] = lg[...] + part

    @pl.when(i == o4)
    def _fin():
        v3row = v3_ref[...].astype(jnp.bfloat16).astype(jnp.float32)
        hv = h2v[...].astype(jnp.float32)
        value_ref[...] = (jnp.sum(hv * v3row, axis=-1, keepdims=True)
                          + vb3_ref[...])
        logits = lg[...] + b3_ref[...]
        m = jnp.max(logits, axis=-1, keepdims=True)
        e = jnp.exp(logits - m)
        p = e / jnp.sum(e, axis=-1, keepdims=True)
        probs_ref[...] = p
        lp = jnp.log(p + 1e-20)
        y = lp + g_ref[...]
        ym = jnp.max(y, axis=-1, keepdims=True)
        cols = jax.lax.broadcasted_iota(jnp.int32, (B, A), 1)
        idx = jnp.min(jnp.where(y == ym, cols, A), axis=-1, keepdims=True)
        act_ref[...] = idx
        alp_ref[...] = jnp.sum(jnp.where(cols == idx, lp, 0.0),
                               axis=-1, keepdims=True)


def kernel(state, W1, b1, W2, b2, W3, b3, V1, vb1, V2, vb2, V3, vb3):
    B, S = state.shape
    H = W1.shape[1]
    A = W3.shape[1]
    nk = S // _KB
    n3 = S // _K3
    o2, o3 = nk, 2 * nk
    o4 = o3 + n3
    steps = o4 + 1

    with jax.ensure_compile_time_eval():
        g = jax.random.gumbel(jax.random.key(42), (B, A), jnp.float32)

    in_specs = [
        pl.BlockSpec((B, S), lambda i: (0, 0)),
        pl.BlockSpec((_KB, H), lambda i: (jnp.clip(i, 0, nk - 1), 0)),
        pl.BlockSpec((1, H), lambda i: (0, 0)),
        pl.BlockSpec((_KB, H), lambda i: (jnp.clip(i - o2, 0, nk - 1), 0)),
        pl.BlockSpec((1, H), lambda i: (0, 0)),
        pl.BlockSpec((_K3, A), lambda i: (jnp.clip(i - o3, 0, n3 - 1), 0)),
        pl.BlockSpec((1, A), lambda i: (0, 0)),
        pl.BlockSpec((_KB, H), lambda i: (jnp.clip(i, 0, nk - 1), 0)),
        pl.BlockSpec((1, H), lambda i: (0, 0)),
        pl.BlockSpec((_KB, H), lambda i: (jnp.clip(i - o2, 0, nk - 1), 0)),
        pl.BlockSpec((1, H), lambda i: (0, 0)),
        pl.BlockSpec((1, S), lambda i: (0, 0)),
        pl.BlockSpec((1, 1), lambda i: (0, 0)),
        pl.BlockSpec((B, A), lambda i: (0, 0)),
    ]
    out_specs = [
        pl.BlockSpec((B, A), lambda i: (0, 0)),
        pl.BlockSpec((B, 1), lambda i: (0, 0)),
        pl.BlockSpec((B, 1), lambda i: (0, 0)),
        pl.BlockSpec((B, 1), lambda i: (0, 0)),
    ]
    out_shape = [
        jax.ShapeDtypeStruct((B, A), jnp.float32),
        jax.ShapeDtypeStruct((B, 1), jnp.float32),
        jax.ShapeDtypeStruct((B, 1), jnp.int32),
        jax.ShapeDtypeStruct((B, 1), jnp.float32),
    ]
    scratch_shapes = [
        pltpu.VMEM((B, S), jnp.bfloat16),
        pltpu.VMEM((B, H), jnp.bfloat16),
        pltpu.VMEM((B, H), jnp.bfloat16),
        pltpu.VMEM((B, H), jnp.bfloat16),
        pltpu.VMEM((B, H), jnp.bfloat16),
        pltpu.VMEM((B, H), jnp.float32),
        pltpu.VMEM((B, H), jnp.float32),
        pltpu.VMEM((B, A), jnp.float32),
    ]

    probs, value, act, alp = pl.pallas_call(
        _body,
        grid=(steps,),
        in_specs=in_specs,
        out_specs=out_specs,
        out_shape=out_shape,
        scratch_shapes=scratch_shapes,
    )(state, W1, b1.reshape(1, H), W2, b2.reshape(1, H),
      W3, b3.reshape(1, A), V1, vb1.reshape(1, H), V2, vb2.reshape(1, H),
      V3.reshape(1, S), vb3.reshape(1, 1), g)
    return probs, value, act[:, 0], alp[:, 0]
